# baseline (speedup 1.0000x reference)
import jax
import jax.numpy as jnp
from jax import lax
from jax.experimental import pallas as pl
from jax.experimental.pallas import tpu as pltpu
from jax.experimental.pallas import tpu_sc as plsc

NUM_ROWS = 1000000
D = 64
BATCH = 16384

NC = 2
NS = 16
NW = NC * NS
B_PER_W = BATCH // NW
L = 16
DC = D // L
NG = B_PER_W // L


def _lane_sum16(v):
    io = lax.iota(jnp.int32, L)
    dn = lax.GatherDimensionNumbers(
        offset_dims=(), collapsed_slice_dims=(0,), start_index_map=(0,))
    for k in (8, 4, 2, 1):
        idx = lax.bitwise_xor(io, jnp.full((L,), k, dtype=jnp.int32))
        v = v + lax.gather(v, idx[:, None], dn, (1,),
                           mode=lax.GatherScatterMode.PROMISE_IN_BOUNDS)
    return v


def _rsqrt16(x):
    i = plsc.bitcast(x, jnp.int32)
    y = plsc.bitcast(
        jnp.full((L,), 0x5F3759DF, dtype=jnp.int32)
        - lax.shift_right_logical(i, jnp.full((L,), 1, dtype=jnp.int32)),
        jnp.float32,
    )
    half = x * 0.5
    for _ in range(3):
        y = y * (1.5 - half * y * y)
    return y


def _body(left_hbm, right_hbm, table_hbm, pos_l_hbm, pos_r_hbm, out_hbm,
          idx_l, idx_r, blk_l, blk_r, out_v, pos_v, sem, sem2):
    wid = lax.axis_index("s") * NC + lax.axis_index("c")
    base = wid * B_PER_W

    pltpu.sync_copy(left_hbm.at[pl.ds(base, B_PER_W)], idx_l)
    pltpu.sync_copy(right_hbm.at[pl.ds(base, B_PER_W)], idx_r)
    pltpu.sync_copy(pos_l_hbm, pos_v.at[0])
    pltpu.sync_copy(pos_r_hbm, pos_v.at[1])
    psum = [pos_v[0, pl.ds(c * L, L)] + pos_v[1, pl.ds(c * L, L)]
            for c in range(DC)]

    def fire(g, buf, s):
        vl = idx_l[pl.ds(g * L, L)]
        vr = idx_r[pl.ds(g * L, L)]
        half_rows = NUM_ROWS // 2
        hl = vl // half_rows
        sl = ((vl - hl * half_rows) // 8) * 8
        hr = vr // half_rows
        sr = ((vr - hr * half_rows) // 8) * 8
        for k in range(L):
            pltpu.make_async_copy(
                table_hbm.at[hl[k], pl.ds(pl.multiple_of(sl[k], 8), 8), :],
                blk_l.at[buf, k], s).start()
        for k in range(L):
            pltpu.make_async_copy(
                table_hbm.at[hr[k], pl.ds(pl.multiple_of(sr[k], 8), 8), :],
                blk_r.at[buf, k], s).start()

    fire(0, 0, sem)
    fire(1, 1, sem2)

    def half_grp(g, buf, s):
        pltpu.make_async_copy(
            table_hbm.at[0, pl.ds(0, L * 8), :], blk_l.at[buf], s).wait()
        pltpu.make_async_copy(
            table_hbm.at[0, pl.ds(0, L * 8), :], blk_r.at[buf], s).wait()

        subl = idx_l[pl.ds(g * L, L)] % 8
        subr = idx_r[pl.ds(g * L, L)] % 8
        for k in range(L):
            sl = subl[k]
            sr = subr[k]
            v = [blk_l[buf, k, sl, pl.ds(c * L, L)]
                 + blk_r[buf, k, sr, pl.ds(c * L, L)] + psum[c]
                 for c in range(DC)]
            ss = v[0] * v[0]
            for c in range(1, DC):
                ss = ss + v[c] * v[c]
            tot = _lane_sum16(ss)
            rinv = _rsqrt16(jnp.maximum(tot, 1e-24))
            row2 = g * (L // 2) + (k // 2)
            off = (k % 2) * D
            for c in range(DC):
                out_v[row2, pl.ds(off + c * L, L)] = v[c] * rinv

        @pl.when(g < NG - 2)
        def _():
            fire(g + 2, buf, s)

    def grp(i, carry):
        half_grp(2 * i, 0, sem)
        half_grp(2 * i + 1, 1, sem2)
        return carry

    lax.fori_loop(0, NG // 2, grp, 0)

    pltpu.sync_copy(out_v, out_hbm.at[pl.ds(wid * (B_PER_W // 2),
                                            B_PER_W // 2)])


@jax.jit
def kernel(left_idx, right_idx, class_emb, pos_left, pos_right):
    mesh = plsc.VectorSubcoreMesh(core_axis_name="c", subcore_axis_name="s")
    run = pl.kernel(
        _body,
        out_type=jax.ShapeDtypeStruct((BATCH // 2, 2 * D), jnp.float32),
        mesh=mesh,
        compiler_params=pltpu.CompilerParams(
            needs_layout_passes=False, use_tc_tiling_on_sc=True),
        scratch_types=[
            pltpu.VMEM((B_PER_W,), jnp.int32),
            pltpu.VMEM((B_PER_W,), jnp.int32),
            pltpu.VMEM((2, L, 8, D), jnp.float32),
            pltpu.VMEM((2, L, 8, D), jnp.float32),
            pltpu.VMEM((B_PER_W // 2, 2 * D), jnp.float32),
            pltpu.VMEM((2, D), jnp.float32),
            pltpu.SemaphoreType.DMA,
            pltpu.SemaphoreType.DMA,
        ],
    )
    out2 = run(left_idx.astype(jnp.int32), right_idx.astype(jnp.int32),
               class_emb.reshape(2, NUM_ROWS // 2, D), pos_left, pos_right)
    return out2.reshape(BATCH, D)
